# Initial kernel scaffold; baseline (speedup 1.0000x reference)
import jax
import jax.numpy as jnp
from jax.experimental import pallas as pl


def kernel(input, table, W, b):
    raise NotImplementedError("write your pallas kernel here")


# speedup vs baseline: 64.0054x; 64.0054x over previous
import functools

import jax
import jax.numpy as jnp
from jax import lax
from jax.experimental import pallas as pl
from jax.experimental.pallas import tpu as pltpu
from jax.experimental.pallas import tpu_sc as plsc

_NC = 2
_NS = 16
_NW = _NC * _NS


def _sdiff_body(tab_ref, wd_ref, bd_ref, s_ref):
    x = tab_ref[...]
    w = wd_ref[...]
    s_ref[...] = jnp.sum(x * w, axis=1, keepdims=True) + bd_ref[0, 0]


@functools.lru_cache(maxsize=None)
def _make_sdiff(V, D, blk):
    return pl.pallas_call(
        _sdiff_body,
        grid=(V // blk,),
        in_specs=[
            pl.BlockSpec((blk, D), lambda i: (i, 0)),
            pl.BlockSpec((1, D), lambda i: (0, 0)),
            pl.BlockSpec(memory_space=pltpu.SMEM),
        ],
        out_specs=pl.BlockSpec((blk, 1), lambda i: (i, 0)),
        out_shape=jax.ShapeDtypeStruct((V, 1), jnp.float32),
    )


@functools.lru_cache(maxsize=None)
def _make_gather(V, N, CH):
    NB = N // _NW
    NCH = NB // CH
    mesh = plsc.VectorSubcoreMesh(core_axis_name="c", subcore_axis_name="s")

    @functools.partial(
        pl.kernel,
        out_type=jax.ShapeDtypeStruct((2 * N,), jnp.float32),
        mesh=mesh,
        compiler_params=pltpu.CompilerParams(needs_layout_passes=False),
        scratch_types=[
            pltpu.VMEM((V,), jnp.float32),
            pltpu.VMEM((CH,), jnp.int32),
            pltpu.VMEM((2 * CH,), jnp.float32),
        ],
    )
    def gather_k(s_hbm, idx_hbm, z_hbm, s_v, idx_v, z_v):
        wid = lax.axis_index("s") * _NC + lax.axis_index("c")
        base = wid * NB
        pltpu.sync_copy(s_hbm, s_v)

        def chunk(c, carry):
            cbase = base + c * CH
            pltpu.sync_copy(idx_hbm.at[pl.ds(cbase, CH)], idx_v)

            def vec(j, carry2):
                iv = idx_v[pl.ds(j * 16, 16)]
                vals = plsc.load_gather(s_v, [iv])
                pos = lax.iota(jnp.int32, 16) + j * 16
                plsc.store_scatter(z_v, [2 * pos], -vals)
                plsc.store_scatter(z_v, [2 * pos + 1], vals)
                return carry2

            lax.fori_loop(0, CH // 16, vec, 0)
            pltpu.sync_copy(z_v, z_hbm.at[pl.ds(2 * cbase, 2 * CH)])
            return carry

        lax.fori_loop(0, NCH, chunk, 0)

    return gather_k


def _softplus_body(z_ref, o_ref):
    z = z_ref[...]
    o_ref[...] = -(jnp.maximum(z, 0.0) + jnp.log1p(jnp.exp(-jnp.abs(z))))


@functools.lru_cache(maxsize=None)
def _make_softplus(R, C, blk):
    return pl.pallas_call(
        _softplus_body,
        grid=(R // blk,),
        in_specs=[pl.BlockSpec((blk, C), lambda i: (i, 0))],
        out_specs=pl.BlockSpec((blk, C), lambda i: (i, 0)),
        out_shape=jax.ShapeDtypeStruct((R, C), jnp.float32),
    )


def kernel(input, table, W, b):
    B, L = input.shape
    V, D = table.shape
    N = B * L

    wd = (W[0] - W[1]).reshape(1, D).astype(jnp.float32)
    bd = (b[0] - b[1]).reshape(1, 1).astype(jnp.float32)

    s = _make_sdiff(V, D, 2000)(table, wd, bd)
    z = _make_gather(V, N, 3200)(
        s.reshape(V), input.reshape(N).astype(jnp.int32))
    o = _make_softplus(B, 2 * L, 512)(z.reshape(B, 2 * L))
    return o.reshape(B, L, 2)


# speedup vs baseline: 106.1389x; 1.6583x over previous
import functools

import jax
import jax.numpy as jnp
from jax import lax
from jax.experimental import pallas as pl
from jax.experimental.pallas import tpu as pltpu
from jax.experimental.pallas import tpu_sc as plsc

_NC = 2
_NS = 16
_NW = _NC * _NS


def _sdiff_body(tab_ref, wd_ref, bd_ref, s_ref):
    x = tab_ref[...]
    w = wd_ref[...]
    s_ref[...] = jnp.sum(x * w, axis=0) + bd_ref[0, 0]


@functools.lru_cache(maxsize=None)
def _make_sdiff(V, D, blk):
    return pl.pallas_call(
        _sdiff_body,
        grid=(pl.cdiv(V, blk),),
        in_specs=[
            pl.BlockSpec((D, blk), lambda i: (0, i)),
            pl.BlockSpec((D, 1), lambda i: (0, 0)),
            pl.BlockSpec(memory_space=pltpu.SMEM),
        ],
        out_specs=pl.BlockSpec((blk,), lambda i: (i,)),
        out_shape=jax.ShapeDtypeStruct((V,), jnp.float32),
    )


@functools.lru_cache(maxsize=None)
def _make_gather(V, N, CH):
    NB = N // _NW
    NCH = NB // CH
    mesh = plsc.VectorSubcoreMesh(core_axis_name="c", subcore_axis_name="s")

    @functools.partial(
        pl.kernel,
        out_type=(jax.ShapeDtypeStruct((N,), jnp.float32),
                  jax.ShapeDtypeStruct((N,), jnp.float32)),
        mesh=mesh,
        compiler_params=pltpu.CompilerParams(needs_layout_passes=False),
        scratch_types=[
            pltpu.VMEM((V,), jnp.float32),
            pltpu.VMEM((CH,), jnp.int32),
            pltpu.VMEM((CH,), jnp.float32),
            pltpu.VMEM((CH,), jnp.float32),
        ],
    )
    def gather_k(s_hbm, idx_hbm, o0_hbm, o1_hbm, s_v, idx_v, o0_v, o1_v):
        wid = lax.axis_index("s") * _NC + lax.axis_index("c")
        base = wid * NB
        pltpu.sync_copy(s_hbm, s_v)

        def chunk(c, carry):
            cbase = base + c * CH
            pltpu.sync_copy(idx_hbm.at[pl.ds(cbase, CH)], idx_v)

            def vec(j, carry2):
                iv = idx_v[pl.ds(j * 16, 16)]
                vals = plsc.load_gather(s_v, [iv])
                m = jnp.maximum(vals, 0.0)
                w = jnp.exp(-jnp.abs(vals))
                t = w / (w + 2.0)
                t2 = t * t
                log1p_w = 2.0 * t * (1.0 + t2 * (
                    0.333333333 + t2 * (0.2 + t2 * 0.142857143)))
                u1 = -(m + log1p_w)
                u0 = u1 + vals
                o0_v[pl.ds(j * 16, 16)] = u0
                o1_v[pl.ds(j * 16, 16)] = u1
                return carry2

            lax.fori_loop(0, CH // 16, vec, 0)
            pltpu.sync_copy(o0_v, o0_hbm.at[pl.ds(cbase, CH)])
            pltpu.sync_copy(o1_v, o1_hbm.at[pl.ds(cbase, CH)])
            return carry

        lax.fori_loop(0, NCH, chunk, 0)

    return gather_k


def kernel(input, table, W, b):
    B, L = input.shape
    V, D = table.shape
    N = B * L

    wd = (W[0] - W[1]).reshape(D, 1).astype(jnp.float32)
    bd = (b[0] - b[1]).reshape(1, 1).astype(jnp.float32)

    s = _make_sdiff(V, D, 8192)(table.T, wd, bd)
    o0, o1 = _make_gather(V, N, 3200)(s, input.reshape(N).astype(jnp.int32))
    return jnp.stack([o0.reshape(B, L), o1.reshape(B, L)], axis=-1)


# speedup vs baseline: 153.5757x; 1.4469x over previous
import functools

import jax
import jax.numpy as jnp
from jax import lax
from jax.experimental import pallas as pl
from jax.experimental.pallas import tpu as pltpu
from jax.experimental.pallas import tpu_sc as plsc

_NC = 2
_NS = 16
_NW = _NC * _NS


def _sdiff_body(tab_ref, wd_ref, bd_ref, s_ref):
    x = tab_ref[...]
    w = wd_ref[...]
    s_ref[...] = jnp.sum(x * w, axis=0) + bd_ref[0, 0]


@functools.lru_cache(maxsize=None)
def _make_sdiff(V, D, blk):
    return pl.pallas_call(
        _sdiff_body,
        grid=(pl.cdiv(V, blk),),
        in_specs=[
            pl.BlockSpec((D, blk), lambda i: (0, i)),
            pl.BlockSpec((D, 1), lambda i: (0, 0)),
            pl.BlockSpec(memory_space=pltpu.SMEM),
        ],
        out_specs=pl.BlockSpec((blk,), lambda i: (i,)),
        out_shape=jax.ShapeDtypeStruct((V,), jnp.float32),
    )


@functools.lru_cache(maxsize=None)
def _make_gather(V, N, CH):
    NB = N // _NW
    NCH = NB // CH
    mesh = plsc.VectorSubcoreMesh(core_axis_name="c", subcore_axis_name="s")

    @functools.partial(
        pl.kernel,
        out_type=jax.ShapeDtypeStruct((N,), jnp.float32),
        mesh=mesh,
        compiler_params=pltpu.CompilerParams(needs_layout_passes=False),
        scratch_types=[
            pltpu.VMEM((V,), jnp.float32),
            pltpu.VMEM((CH,), jnp.int32),
            pltpu.VMEM((CH,), jnp.float32),
        ],
    )
    def gather_k(s_hbm, idx_hbm, g_hbm, s_v, idx_v, g_v):
        wid = lax.axis_index("s") * _NC + lax.axis_index("c")
        base = wid * NB
        pltpu.sync_copy(s_hbm, s_v)

        def chunk(c, carry):
            cbase = base + c * CH
            pltpu.sync_copy(idx_hbm.at[pl.ds(cbase, CH)], idx_v)

            def vec(j, carry2):
                iv = idx_v[pl.ds(j * 16, 16)]
                g_v[pl.ds(j * 16, 16)] = plsc.load_gather(s_v, [iv])
                return carry2

            lax.fori_loop(0, CH // 16, vec, 0)
            pltpu.sync_copy(g_v, g_hbm.at[pl.ds(cbase, CH)])
            return carry

        lax.fori_loop(0, NCH, chunk, 0)

    return gather_k


def _softplus_body(g_ref, o_ref):
    g = g_ref[...]
    u1 = -(jnp.maximum(g, 0.0) + jnp.log1p(jnp.exp(-jnp.abs(g))))
    o_ref[0] = u1 + g
    o_ref[1] = u1


@functools.lru_cache(maxsize=None)
def _make_softplus(B, L, blk):
    return pl.pallas_call(
        _softplus_body,
        grid=(B // blk,),
        in_specs=[pl.BlockSpec((blk, L), lambda i: (i, 0))],
        out_specs=pl.BlockSpec((2, blk, L), lambda i: (0, i, 0)),
        out_shape=jax.ShapeDtypeStruct((2, B, L), jnp.float32),
    )


def kernel(input, table, W, b):
    B, L = input.shape
    V, D = table.shape
    N = B * L

    wd = (W[0] - W[1]).reshape(D, 1).astype(jnp.float32)
    bd = (b[0] - b[1]).reshape(1, 1).astype(jnp.float32)

    s = _make_sdiff(V, D, 8192)(table.T, wd, bd)
    g = _make_gather(V, N, 3200)(s, input.reshape(N).astype(jnp.int32))
    o = _make_softplus(B, L, 512)(g.reshape(B, L))
    return o.transpose(1, 2, 0)


# speedup vs baseline: 177.6088x; 1.1565x over previous
import functools

import jax
import jax.numpy as jnp
from jax import lax
from jax.experimental import pallas as pl
from jax.experimental.pallas import tpu as pltpu
from jax.experimental.pallas import tpu_sc as plsc

_NC = 2
_NS = 16
_NW = _NC * _NS


def _sdiff_body(tab_ref, wd_ref, bd_ref, s_ref):
    x = tab_ref[...]
    w = wd_ref[...]
    s_ref[...] = jnp.sum(x * w, axis=0) + bd_ref[0, 0]


@functools.lru_cache(maxsize=None)
def _make_sdiff(V, D, blk):
    return pl.pallas_call(
        _sdiff_body,
        grid=(pl.cdiv(V, blk),),
        in_specs=[
            pl.BlockSpec((D, blk), lambda i: (0, i)),
            pl.BlockSpec((D, 1), lambda i: (0, 0)),
            pl.BlockSpec(memory_space=pltpu.SMEM),
        ],
        out_specs=pl.BlockSpec((blk,), lambda i: (i,)),
        out_shape=jax.ShapeDtypeStruct((V,), jnp.float32),
    )


@functools.lru_cache(maxsize=None)
def _make_gather(V, N, CH):
    NB = N // _NW
    NCH = NB // CH
    mesh = plsc.VectorSubcoreMesh(core_axis_name="c", subcore_axis_name="s")

    @functools.partial(
        pl.kernel,
        out_type=jax.ShapeDtypeStruct((2 * N,), jnp.float32),
        mesh=mesh,
        compiler_params=pltpu.CompilerParams(needs_layout_passes=False),
        scratch_types=[
            pltpu.VMEM((V,), jnp.float32),
            pltpu.VMEM((CH,), jnp.int32),
            pltpu.VMEM((2 * CH,), jnp.float32),
        ],
    )
    def gather_k(s_hbm, idx_hbm, g_hbm, s_v, idx_v, g_v):
        wid = lax.axis_index("s") * _NC + lax.axis_index("c")
        base = wid * NB
        pltpu.sync_copy(s_hbm, s_v)

        def chunk(c, carry):
            cbase = base + c * CH
            pltpu.sync_copy(idx_hbm.at[pl.ds(cbase, CH)], idx_v)

            def vec(j, carry2):
                iv = idx_v[pl.ds(j * 16, 16)]
                vals = plsc.load_gather(s_v, [iv])
                d1 = (j // 8) * 256 + (j % 8) * 16
                g_v[pl.ds(d1, 16)] = vals
                g_v[pl.ds(d1 + 128, 16)] = vals
                return carry2

            lax.fori_loop(0, CH // 16, vec, 0)
            pltpu.sync_copy(g_v, g_hbm.at[pl.ds(2 * cbase, 2 * CH)])
            return carry

        lax.fori_loop(0, NCH, chunk, 0)

    return gather_k


def _softplus_body(g_ref, o_ref):
    g = g_ref[...]
    u1 = -(jnp.maximum(g, 0.0) + jnp.log1p(jnp.exp(-jnp.abs(g))))
    k0 = lax.broadcasted_iota(jnp.int32, g.shape, 0) % 2 == 0
    o_ref[...] = u1 + jnp.where(k0, g, 0.0)


@functools.lru_cache(maxsize=None)
def _make_softplus(R, blk):
    return pl.pallas_call(
        _softplus_body,
        grid=(R // blk,),
        in_specs=[pl.BlockSpec((blk, 128), lambda i: (i, 0))],
        out_specs=pl.BlockSpec((blk, 128), lambda i: (i, 0)),
        out_shape=jax.ShapeDtypeStruct((R, 128), jnp.float32),
    )


def kernel(input, table, W, b):
    B, L = input.shape
    V, D = table.shape
    N = B * L
    NJ = B // 128

    wd = (W[0] - W[1]).reshape(D, 1).astype(jnp.float32)
    bd = (b[0] - b[1]).reshape(1, 1).astype(jnp.float32)

    s = _make_sdiff(V, D, 8192)(table.T, wd, bd)
    idx_lm = input.T.reshape(N).astype(jnp.int32)
    g2 = _make_gather(V, N, 3200)(s, idx_lm)
    o = _make_softplus(2 * N // 128, 1600)(g2.reshape(2 * N // 128, 128))
    return (o.reshape(L, NJ, 2, 128).transpose(1, 3, 0, 2).reshape(B, L, 2))


# speedup vs baseline: 201.0106x; 1.1318x over previous
import functools

import jax
import jax.numpy as jnp
from jax import lax
from jax.experimental import pallas as pl
from jax.experimental.pallas import tpu as pltpu
from jax.experimental.pallas import tpu_sc as plsc

_NC = 2
_NS = 16
_NW = _NC * _NS


def _sdiff_body(tab_ref, wd_ref, bd_ref, s_ref):
    x = tab_ref[...]
    w = wd_ref[...]
    s_ref[...] = jnp.sum(x * w, axis=0) + bd_ref[0, 0]


@functools.lru_cache(maxsize=None)
def _make_sdiff(V, D, blk):
    return pl.pallas_call(
        _sdiff_body,
        grid=(pl.cdiv(V, blk),),
        in_specs=[
            pl.BlockSpec((D, blk), lambda i: (0, i)),
            pl.BlockSpec((D, 1), lambda i: (0, 0)),
            pl.BlockSpec(memory_space=pltpu.SMEM),
        ],
        out_specs=pl.BlockSpec((blk,), lambda i: (i,)),
        out_shape=jax.ShapeDtypeStruct((V,), jnp.float32),
    )


@functools.lru_cache(maxsize=None)
def _make_gather(V, N, CH):
    NB = N // _NW
    NCH = NB // CH
    mesh = plsc.VectorSubcoreMesh(core_axis_name="c", subcore_axis_name="s")

    @functools.partial(
        pl.kernel,
        out_type=jax.ShapeDtypeStruct((2 * N,), jnp.float32),
        mesh=mesh,
        compiler_params=pltpu.CompilerParams(needs_layout_passes=False),
        scratch_types=[
            pltpu.VMEM((V,), jnp.float32),
            pltpu.VMEM((CH,), jnp.int32),
            pltpu.VMEM((2 * CH,), jnp.float32),
        ],
    )
    def gather_k(s_hbm, idx_hbm, g_hbm, s_v, idx_v, g_v):
        wid = lax.axis_index("s") * _NC + lax.axis_index("c")
        base = wid * NB
        pltpu.sync_copy(s_hbm, s_v)

        def chunk(c, carry):
            cbase = base + c * CH
            pltpu.sync_copy(idx_hbm.at[pl.ds(cbase, CH)], idx_v)

            @plsc.parallel_loop(0, CH // 128, unroll=2)
            def vec(gi):
                for p in range(8):
                    iv = idx_v[pl.ds(gi * 128 + p * 16, 16)]
                    vals = plsc.load_gather(s_v, [iv])
                    d1 = gi * 256 + p * 16
                    g_v[pl.ds(d1, 16)] = vals
                    g_v[pl.ds(d1 + 128, 16)] = vals
            pltpu.sync_copy(g_v, g_hbm.at[pl.ds(2 * cbase, 2 * CH)])
            return carry

        lax.fori_loop(0, NCH, chunk, 0)

    return gather_k


def _softplus_body(g_ref, o_ref):
    g = g_ref[...]
    u1 = -(jnp.maximum(g, 0.0) + jnp.log1p(jnp.exp(-jnp.abs(g))))
    k0 = lax.broadcasted_iota(jnp.int32, g.shape, 0) % 2 == 0
    o_ref[...] = u1 + jnp.where(k0, g, 0.0)


@functools.lru_cache(maxsize=None)
def _make_softplus(R, blk):
    return pl.pallas_call(
        _softplus_body,
        grid=(R // blk,),
        in_specs=[pl.BlockSpec((blk, 128), lambda i: (i, 0))],
        out_specs=pl.BlockSpec((blk, 128), lambda i: (i, 0)),
        out_shape=jax.ShapeDtypeStruct((R, 128), jnp.float32),
    )


def kernel(input, table, W, b):
    B, L = input.shape
    V, D = table.shape
    N = B * L
    NJ = B // 128

    wd = (W[0] - W[1]).reshape(D, 1).astype(jnp.float32)
    bd = (b[0] - b[1]).reshape(1, 1).astype(jnp.float32)

    s = _make_sdiff(V, D, 8192)(table.T, wd, bd)
    idx_lm = input.T.reshape(N).astype(jnp.int32)
    g2 = _make_gather(V, N, 3200)(s, idx_lm)
    o = _make_softplus(2 * N // 128, 1600)(g2.reshape(2 * N // 128, 128))
    return (o.reshape(L, NJ, 2, 128).transpose(1, 3, 0, 2).reshape(B, L, 2))


# speedup vs baseline: 214.7585x; 1.0684x over previous
import functools

import jax
import jax.numpy as jnp
from jax import lax
from jax.experimental import pallas as pl
from jax.experimental.pallas import tpu as pltpu
from jax.experimental.pallas import tpu_sc as plsc

_NC = 2
_NS = 16
_NW = _NC * _NS


def _sdiff_body(tab_ref, wd_ref, bd_ref, s_ref):
    x = tab_ref[...]
    w = wd_ref[...]
    s_ref[...] = jnp.sum(x * w, axis=0) + bd_ref[0, 0]


@functools.lru_cache(maxsize=None)
def _make_sdiff(V, D, blk):
    return pl.pallas_call(
        _sdiff_body,
        grid=(pl.cdiv(V, blk),),
        in_specs=[
            pl.BlockSpec((D, blk), lambda i: (0, i)),
            pl.BlockSpec((D, 1), lambda i: (0, 0)),
            pl.BlockSpec(memory_space=pltpu.SMEM),
        ],
        out_specs=pl.BlockSpec((blk,), lambda i: (i,)),
        out_shape=jax.ShapeDtypeStruct((V,), jnp.float32),
    )


@functools.lru_cache(maxsize=None)
def _make_gather(V, N, CH, NJ):
    NB = N // _NW
    NCH = NB // CH
    SPC = CH // 1024
    mesh = plsc.VectorSubcoreMesh(core_axis_name="c", subcore_axis_name="s")

    @functools.partial(
        pl.kernel,
        out_type=jax.ShapeDtypeStruct((2 * N,), jnp.float32),
        mesh=mesh,
        compiler_params=pltpu.CompilerParams(needs_layout_passes=False),
        scratch_types=[
            pltpu.VMEM((V,), jnp.float32),
            pltpu.VMEM((CH,), jnp.int32),
            pltpu.VMEM((2048,), jnp.float32),
            pltpu.VMEM((2048,), jnp.float32),
            pltpu.SemaphoreType.DMA,
            pltpu.SemaphoreType.DMA,
        ],
    )
    def gather_k(s_hbm, idx_hbm, g_hbm, s_v, idx_v, g0_v, g1_v, sem0, sem1):
        wid = lax.axis_index("s") * _NC + lax.axis_index("c")
        base = wid * NB
        pltpu.sync_copy(s_hbm, s_v)
        g_bufs = (g0_v, g1_v)
        sems = (sem0, sem1)

        def chunk(c, carry):
            pltpu.sync_copy(idx_hbm.at[pl.ds(base + c * CH, CH)], idx_v)
            for sub in range(SPC):
                par = sub % 2
                g_v, sem = g_bufs[par], sems[par]
                scc = c * SPC + sub
                G0 = wid * (NB // 128) + scc * 8
                lt = G0 // (8 * NJ)
                bt = (G0 // 8) % NJ
                obase = lt * (2048 * NJ) + bt * 256

                @pl.when(scc >= 2)
                def _drain():
                    for lp in range(8):
                        pltpu.make_async_copy(
                            g_v.at[pl.ds(lp * 256, 256)],
                            g_hbm.at[pl.ds(lp * 256, 256)], sem).wait()

                @plsc.parallel_loop(0, 8, unroll=2)
                def vec(lp):
                    for p in range(8):
                        iv = idx_v[pl.ds(sub * 1024 + lp * 128 + p * 16, 16)]
                        vals = plsc.load_gather(s_v, [iv])
                        g_v[pl.ds(lp * 256 + p * 16, 16)] = vals
                        g_v[pl.ds(lp * 256 + 128 + p * 16, 16)] = vals

                for lp in range(8):
                    pltpu.async_copy(
                        g_v.at[pl.ds(lp * 256, 256)],
                        g_hbm.at[pl.ds(obase + lp * (256 * NJ), 256)],
                        sem)
            return carry

        lax.fori_loop(0, NCH, chunk, 0)
        for par in range(2):
            g_v, sem = g_bufs[par], sems[par]
            for lp in range(8):
                pltpu.make_async_copy(
                    g_v.at[pl.ds(lp * 256, 256)],
                    g_hbm.at[pl.ds(lp * 256, 256)], sem).wait()

    return gather_k


def _softplus_body(g_ref, o_ref):
    g = g_ref[...]
    u1 = -(jnp.maximum(g, 0.0) + jnp.log1p(jnp.exp(-jnp.abs(g))))
    k0 = lax.broadcasted_iota(jnp.int32, g.shape, 0) % 2 == 0
    o_ref[...] = u1 + jnp.where(k0, g, 0.0)


@functools.lru_cache(maxsize=None)
def _make_softplus(R, blk):
    return pl.pallas_call(
        _softplus_body,
        grid=(R // blk,),
        in_specs=[pl.BlockSpec((blk, 128), lambda i: (i, 0))],
        out_specs=pl.BlockSpec((blk, 128), lambda i: (i, 0)),
        out_shape=jax.ShapeDtypeStruct((R, 128), jnp.float32),
    )


def kernel(input, table, W, b):
    B, L = input.shape
    V, D = table.shape
    N = B * L
    NJ = B // 128

    wd = (W[0] - W[1]).reshape(D, 1).astype(jnp.float32)
    bd = (b[0] - b[1]).reshape(1, 1).astype(jnp.float32)

    LT = L // 8
    s = _make_sdiff(V, D, 8192)(table.T, wd, bd)
    idx_raw = (input.T.reshape(LT, 8, NJ, 128).transpose(0, 2, 1, 3)
               .reshape(N).astype(jnp.int32))
    g2 = _make_gather(V, N, 5120, NJ)(s, idx_raw)
    o = _make_softplus(2 * N // 128, 1600)(g2.reshape(2 * N // 128, 128))
    return (o.reshape(L, NJ, 2, 128).transpose(1, 3, 0, 2).reshape(B, L, 2))


# speedup vs baseline: 215.8707x; 1.0052x over previous
import functools

import jax
import jax.numpy as jnp
from jax import lax
from jax.experimental import pallas as pl
from jax.experimental.pallas import tpu as pltpu
from jax.experimental.pallas import tpu_sc as plsc

_NC = 2
_NS = 16
_NW = _NC * _NS


def _sdiff_body(tab_ref, wd_ref, bd_ref, s_ref):
    x = tab_ref[...]
    w = wd_ref[...]
    s_ref[...] = jnp.sum(x * w, axis=0) + bd_ref[0, 0]


@functools.lru_cache(maxsize=None)
def _make_sdiff(V, D, blk):
    return pl.pallas_call(
        _sdiff_body,
        grid=(pl.cdiv(V, blk),),
        in_specs=[
            pl.BlockSpec((D, blk), lambda i: (0, i)),
            pl.BlockSpec((D, 1), lambda i: (0, 0)),
            pl.BlockSpec(memory_space=pltpu.SMEM),
        ],
        out_specs=pl.BlockSpec((blk,), lambda i: (i,)),
        out_shape=jax.ShapeDtypeStruct((V,), jnp.float32),
    )


@functools.lru_cache(maxsize=None)
def _make_gather(V, N, CH, NJ):
    NB = N // _NW
    NCH = NB // CH
    SPC = CH // 1024
    mesh = plsc.VectorSubcoreMesh(core_axis_name="c", subcore_axis_name="s")

    @functools.partial(
        pl.kernel,
        out_type=jax.ShapeDtypeStruct((N,), jnp.float32),
        mesh=mesh,
        compiler_params=pltpu.CompilerParams(needs_layout_passes=False),
        scratch_types=[
            pltpu.VMEM((V,), jnp.float32),
            pltpu.VMEM((CH,), jnp.int32),
            pltpu.VMEM((1024,), jnp.float32),
            pltpu.VMEM((1024,), jnp.float32),
            pltpu.SemaphoreType.DMA,
            pltpu.SemaphoreType.DMA,
        ],
    )
    def gather_k(s_hbm, idx_hbm, g_hbm, s_v, idx_v, g0_v, g1_v, sem0, sem1):
        wid = lax.axis_index("s") * _NC + lax.axis_index("c")
        base = wid * NB
        pltpu.sync_copy(s_hbm, s_v)
        g_bufs = (g0_v, g1_v)
        sems = (sem0, sem1)

        def chunk(c, carry):
            pltpu.sync_copy(idx_hbm.at[pl.ds(base + c * CH, CH)], idx_v)
            for sub in range(SPC):
                par = sub % 2
                g_v, sem = g_bufs[par], sems[par]
                scc = c * SPC + sub
                G0 = wid * (NB // 128) + scc * 8
                lt = G0 // (8 * NJ)
                bt = (G0 // 8) % NJ
                obase = lt * (1024 * NJ) + bt * 128

                @pl.when(scc >= 2)
                def _drain():
                    for lp in range(8):
                        pltpu.make_async_copy(
                            g_v.at[pl.ds(lp * 128, 128)],
                            g_hbm.at[pl.ds(lp * 128, 128)], sem).wait()

                @plsc.parallel_loop(0, 8, unroll=2)
                def vec(lp):
                    for p in range(8):
                        iv = idx_v[pl.ds(sub * 1024 + lp * 128 + p * 16, 16)]
                        vals = plsc.load_gather(s_v, [iv])
                        g_v[pl.ds(lp * 128 + p * 16, 16)] = vals

                for lp in range(8):
                    pltpu.async_copy(
                        g_v.at[pl.ds(lp * 128, 128)],
                        g_hbm.at[pl.ds(obase + lp * (128 * NJ), 128)],
                        sem)
            return carry

        lax.fori_loop(0, NCH, chunk, 0)
        for par in range(2):
            g_v, sem = g_bufs[par], sems[par]
            for lp in range(8):
                pltpu.make_async_copy(
                    g_v.at[pl.ds(lp * 128, 128)],
                    g_hbm.at[pl.ds(lp * 128, 128)], sem).wait()

    return gather_k


def _softplus_body(g_ref, o_ref):
    g = g_ref[...]
    u1 = -(jnp.maximum(g, 0.0) + jnp.log1p(jnp.exp(-jnp.abs(g))))
    blk = g.shape[0]
    o_ref[...] = jnp.stack([u1 + g, u1], axis=1).reshape(2 * blk, 128)


@functools.lru_cache(maxsize=None)
def _make_softplus(R, blk):
    return pl.pallas_call(
        _softplus_body,
        grid=(R // blk,),
        in_specs=[pl.BlockSpec((blk, 128), lambda i: (i, 0))],
        out_specs=pl.BlockSpec((2 * blk, 128), lambda i: (i, 0)),
        out_shape=jax.ShapeDtypeStruct((2 * R, 128), jnp.float32),
    )


def kernel(input, table, W, b):
    B, L = input.shape
    V, D = table.shape
    N = B * L
    NJ = B // 128

    wd = (W[0] - W[1]).reshape(D, 1).astype(jnp.float32)
    bd = (b[0] - b[1]).reshape(1, 1).astype(jnp.float32)

    LT = L // 8
    s = _make_sdiff(V, D, 8192)(table.T, wd, bd)
    idx_raw = (input.T.reshape(LT, 8, NJ, 128).transpose(0, 2, 1, 3)
               .reshape(N).astype(jnp.int32))
    g1 = _make_gather(V, N, 5120, NJ)(s, idx_raw)
    o = _make_softplus(N // 128, 800)(g1.reshape(N // 128, 128))
    return (o.reshape(L, NJ, 2, 128).transpose(1, 3, 0, 2).reshape(B, L, 2))


# speedup vs baseline: 216.5992x; 1.0034x over previous
import functools

import jax
import jax.numpy as jnp
from jax import lax
from jax.experimental import pallas as pl
from jax.experimental.pallas import tpu as pltpu
from jax.experimental.pallas import tpu_sc as plsc

_NC = 2
_NS = 16
_NW = _NC * _NS


def _sdiff_body(tab_ref, wd_ref, bd_ref, s_ref):
    x = tab_ref[...]
    w = wd_ref[...]
    s_ref[...] = jnp.sum(x * w, axis=0) + bd_ref[0, 0]


@functools.lru_cache(maxsize=None)
def _make_sdiff(V, D, blk):
    return pl.pallas_call(
        _sdiff_body,
        grid=(pl.cdiv(V, blk),),
        in_specs=[
            pl.BlockSpec((D, blk), lambda i: (0, i)),
            pl.BlockSpec((D, 1), lambda i: (0, 0)),
            pl.BlockSpec(memory_space=pltpu.SMEM),
        ],
        out_specs=pl.BlockSpec((blk,), lambda i: (i,)),
        out_shape=jax.ShapeDtypeStruct((V,), jnp.float32),
    )


@functools.lru_cache(maxsize=None)
def _make_gather(V, N, CH, NJ):
    NB = N // _NW
    NCH = NB // CH
    SPC = CH // 1024
    mesh = plsc.VectorSubcoreMesh(core_axis_name="c", subcore_axis_name="s")

    @functools.partial(
        pl.kernel,
        out_type=jax.ShapeDtypeStruct((N,), jnp.float32),
        mesh=mesh,
        compiler_params=pltpu.CompilerParams(needs_layout_passes=False),
        scratch_types=[
            pltpu.VMEM((V,), jnp.float32),
            pltpu.VMEM((CH,), jnp.int32),
            pltpu.VMEM((1024,), jnp.float32),
            pltpu.VMEM((1024,), jnp.float32),
            pltpu.SemaphoreType.DMA,
            pltpu.SemaphoreType.DMA,
        ],
    )
    def gather_k(s_hbm, idx_hbm, g_hbm, s_v, idx_v, g0_v, g1_v, sem0, sem1):
        wid = lax.axis_index("s") * _NC + lax.axis_index("c")
        base = wid * NB
        pltpu.sync_copy(s_hbm, s_v)
        g_bufs = (g0_v, g1_v)
        sems = (sem0, sem1)

        def chunk(c, carry):
            pltpu.sync_copy(idx_hbm.at[pl.ds(base + c * CH, CH)], idx_v)
            for sub in range(SPC):
                par = sub % 2
                g_v, sem = g_bufs[par], sems[par]
                scc = c * SPC + sub
                G0 = wid * (NB // 128) + scc * 8
                lt = G0 // (8 * NJ)
                bt = (G0 // 8) % NJ
                obase = lt * (1024 * NJ) + bt * 128

                @pl.when(scc >= 2)
                def _drain():
                    for lp in range(8):
                        pltpu.make_async_copy(
                            g_v.at[pl.ds(lp * 128, 128)],
                            g_hbm.at[pl.ds(lp * 128, 128)], sem).wait()

                @plsc.parallel_loop(0, 8, unroll=4)
                def vec(lp):
                    for p in range(8):
                        iv = idx_v[pl.ds(sub * 1024 + lp * 128 + p * 16, 16)]
                        vals = plsc.load_gather(s_v, [iv])
                        g_v[pl.ds(lp * 128 + p * 16, 16)] = vals

                for lp in range(8):
                    pltpu.async_copy(
                        g_v.at[pl.ds(lp * 128, 128)],
                        g_hbm.at[pl.ds(obase + lp * (128 * NJ), 128)],
                        sem)
            return carry

        lax.fori_loop(0, NCH, chunk, 0)
        for par in range(2):
            g_v, sem = g_bufs[par], sems[par]
            for lp in range(8):
                pltpu.make_async_copy(
                    g_v.at[pl.ds(lp * 128, 128)],
                    g_hbm.at[pl.ds(lp * 128, 128)], sem).wait()

    return gather_k


def _softplus_body(g_ref, o_ref):
    g = g_ref[...]
    u1 = -(jnp.maximum(g, 0.0) + jnp.log1p(jnp.exp(-jnp.abs(g))))
    blk = g.shape[0]
    o_ref[...] = jnp.stack([u1 + g, u1], axis=1).reshape(2 * blk, 128)


@functools.lru_cache(maxsize=None)
def _make_softplus(R, blk):
    return pl.pallas_call(
        _softplus_body,
        grid=(R // blk,),
        in_specs=[pl.BlockSpec((blk, 128), lambda i: (i, 0))],
        out_specs=pl.BlockSpec((2 * blk, 128), lambda i: (i, 0)),
        out_shape=jax.ShapeDtypeStruct((2 * R, 128), jnp.float32),
    )


def kernel(input, table, W, b):
    B, L = input.shape
    V, D = table.shape
    N = B * L
    NJ = B // 128

    wd = (W[0] - W[1]).reshape(D, 1).astype(jnp.float32)
    bd = (b[0] - b[1]).reshape(1, 1).astype(jnp.float32)

    LT = L // 8
    s = _make_sdiff(V, D, 8192)(table.T, wd, bd)
    idx_raw = (input.T.reshape(LT, 8, NJ, 128).transpose(0, 2, 1, 3)
               .reshape(N).astype(jnp.int32))
    g1 = _make_gather(V, N, 5120, NJ)(s, idx_raw)
    o = _make_softplus(N // 128, 800)(g1.reshape(N // 128, 128))
    return (o.reshape(L, NJ, 2, 128).transpose(1, 3, 0, 2).reshape(B, L, 2))
